# baseline (speedup 1.0000x reference)
import functools

import jax
import jax.numpy as jnp
from jax import lax
from jax.experimental import pallas as pl
from jax.experimental.pallas import tpu as pltpu
from jax.experimental.pallas import tpu_sc as plsc

_NPF = 256
_NPX = 26
_NPY = 230


def _tile_kernel(h, w, apad_ref, pos_ref):
    hw = h * w
    r = jax.lax.broadcasted_iota(jnp.int32, (w + h, hw), 0)
    k = jax.lax.broadcasted_iota(jnp.int32, (w + h, hw), 1)
    ge = jnp.right_shift(r - w, 31) + 1
    v = (k % w) * (1 - ge) + (k // w) * ge
    t = r - w * ge
    b2 = (1 - jnp.minimum(jnp.abs(v - t), 1)).astype(jnp.float32)
    pos_ref[...] = jax.lax.dot_general(
        apad_ref[...], b2, (((0,), (0,)), ((), ())),
        preferred_element_type=jnp.float32,
        precision=jax.lax.Precision.HIGHEST,
    )


def _replicate_body(b, hw, n_workers, pos_hbm, out_hbm, slab, sem):
    ch_per_w = _NPF // n_workers
    wid = lax.axis_index("s") * 2 + lax.axis_index("c")
    base = wid * ch_per_w
    pltpu.sync_copy(pos_hbm.at[pl.ds(base, ch_per_w)], slab)
    copies = [
        pltpu.make_async_copy(slab, out_hbm.at[i, pl.ds(base, ch_per_w)], sem)
        for i in range(b)
    ]
    for c in copies:
        c.start()
    for c in copies:
        c.wait()


def kernel(x, row_embed, col_embed):
    b = x.shape[0]
    h, w = x.shape[-2], x.shape[-1]
    hw = h * w
    ce = col_embed[:w]
    re = row_embed[:h]
    top = jnp.pad(ce, ((0, 0), (0, _NPY)))
    bot = jnp.pad(re, ((0, 0), (_NPX, 0)))
    apad = jnp.concatenate([top, bot], axis=0)

    pos = pl.pallas_call(
        functools.partial(_tile_kernel, h, w),
        out_shape=jax.ShapeDtypeStruct((_NPF, hw), jnp.float32),
    )(apad)

    n_workers = 32
    mesh = plsc.VectorSubcoreMesh(core_axis_name="c", subcore_axis_name="s")
    out = pl.kernel(
        functools.partial(_replicate_body, b, hw, n_workers),
        out_type=jax.ShapeDtypeStruct((b, _NPF, hw), jnp.float32),
        mesh=mesh,
        scratch_types=[
            pltpu.VMEM((_NPF // n_workers, hw), jnp.float32),
            pltpu.SemaphoreType.DMA,
        ],
    )(pos)
    return out.reshape(b, _NPF, h, w)
